# baseline (speedup 1.0000x reference)
import functools

import jax
import jax.numpy as jnp
from jax.experimental import pallas as pl
from jax.experimental.pallas import tpu as pltpu


def _round_up(x, m):
    return ((x + m - 1) // m) * m


def _head_kernel(x_ref, w1t_ref, b1_ref, w2t_ref, b2_ref, out_ref, *, inv_hw):
    x = x_ref[...]
    pooled = jnp.sum(x, axis=(2, 3)) * inv_hw
    h = jnp.dot(pooled, w1t_ref[...],
                preferred_element_type=jnp.float32) + b1_ref[...]
    out = jnp.dot(h, w2t_ref[...],
                  preferred_element_type=jnp.float32) + b2_ref[...]
    out_ref[0] = out.astype(out_ref.dtype)


def kernel(x, w1, b1, w2, b2):
    B, C, H, W = x.shape
    hidden = w1.shape[0]
    NC = w2.shape[0]
    HW = H * W

    TB = 4
    B_pad = _round_up(max(B, TB), TB)
    H_pad = _round_up(hidden, 128)
    NC_pad = _round_up(NC, 128)

    if B_pad != B:
        x = jnp.pad(x, ((0, B_pad - B), (0, 0), (0, 0), (0, 0)))

    w1t = jnp.pad(w1.T, ((0, 0), (0, H_pad - hidden)))
    b1_row = jnp.pad(b1.reshape(1, -1), ((0, 0), (0, H_pad - hidden)))
    w2t = jnp.pad(w2.T, ((0, H_pad - hidden), (0, NC_pad - NC)))
    b2_row = jnp.pad(b2.reshape(1, -1), ((0, 0), (0, NC_pad - NC)))

    n_b = B_pad // TB
    x_tile_bytes = TB * C * _round_up(H, 8) * _round_up(W, 128) * 4
    weight_bytes = (C * H_pad + H_pad + H_pad * NC_pad + NC_pad) * 4
    vmem_limit = min(2 * x_tile_bytes + 2 * weight_bytes
                     + TB * NC_pad * 4 + (8 << 20), 110 << 20)

    cost = pl.CostEstimate(
        flops=B_pad * C * HW + 2 * B_pad * (C * H_pad + H_pad * NC_pad),
        transcendentals=0,
        bytes_accessed=(B_pad * C * HW * 4 + weight_bytes + B_pad * NC_pad * 4),
    )

    out_padded = pl.pallas_call(
        functools.partial(_head_kernel, inv_hw=1.0 / float(HW)),
        out_shape=jax.ShapeDtypeStruct((n_b, TB, NC_pad), jnp.float32),
        grid=(n_b,),
        in_specs=[
            pl.BlockSpec((TB, C, H, W), lambda i: (i, 0, 0, 0)),
            pl.BlockSpec((C, H_pad), lambda i: (0, 0)),
            pl.BlockSpec((1, H_pad), lambda i: (0, 0)),
            pl.BlockSpec((H_pad, NC_pad), lambda i: (0, 0)),
            pl.BlockSpec((1, NC_pad), lambda i: (0, 0)),
        ],
        out_specs=pl.BlockSpec((1, TB, NC_pad), lambda i: (i, 0, 0)),
        compiler_params=pltpu.CompilerParams(
            dimension_semantics=("parallel",),
            vmem_limit_bytes=vmem_limit,
        ),
        cost_estimate=cost,
    )(x, w1t, b1_row, w2t, b2_row)

    return out_padded.reshape(B_pad, NC_pad)[:B, :NC]


# speedup vs baseline: 2.3382x; 2.3382x over previous
import functools

import jax
import jax.numpy as jnp
from jax.experimental import pallas as pl
from jax.experimental.pallas import tpu as pltpu


def _round_up(x, m):
    return ((x + m - 1) // m) * m


def _head_kernel(x_ref, w1t_ref, b1_ref, w2t_ref, b2_ref, out_ref, *, inv_hw):
    x = x_ref[...]
    hw = x.shape[-1]
    n_chunks = hw // 128
    if n_chunks == 0:
        acc = jnp.sum(x.astype(jnp.float32), axis=-1)
    else:
        chunks = [x[:, :, s * 128:(s + 1) * 128].astype(jnp.float32)
                  for s in range(n_chunks)]
        rem = hw - n_chunks * 128
        while len(chunks) > 1:
            nxt = [chunks[i] + chunks[i + 1]
                   for i in range(0, len(chunks) - 1, 2)]
            if len(chunks) % 2:
                nxt.append(chunks[-1])
            chunks = nxt
        acc = jnp.sum(chunks[0], axis=-1)
        if rem:
            acc = acc + jnp.sum(
                x[:, :, n_chunks * 128:].astype(jnp.float32), axis=-1)
    pooled = acc * inv_hw

    h = jnp.dot(pooled, w1t_ref[...],
                preferred_element_type=jnp.float32) + b1_ref[...]
    out = jnp.dot(h, w2t_ref[...],
                  preferred_element_type=jnp.float32) + b2_ref[...]
    out_ref[...] = out.astype(out_ref.dtype)


def kernel(x, w1, b1, w2, b2):
    B, C, H, W = x.shape
    hidden = w1.shape[0]
    NC = w2.shape[0]
    HW = H * W

    TB = 8
    B_pad = _round_up(max(B, TB), TB)
    H_pad = _round_up(hidden, 128)
    NC_pad = _round_up(NC, 128)

    xr = x.astype(jnp.bfloat16).reshape(B, C, HW)
    if B_pad != B:
        xr = jnp.pad(xr, ((0, B_pad - B), (0, 0), (0, 0)))

    w1t = jnp.pad(w1.T, ((0, 0), (0, H_pad - hidden)))
    b1_row = jnp.pad(b1.reshape(1, -1), ((0, 0), (0, H_pad - hidden)))
    w2t = jnp.pad(w2.T, ((0, H_pad - hidden), (0, NC_pad - NC)))
    b2_row = jnp.pad(b2.reshape(1, -1), ((0, 0), (0, NC_pad - NC)))

    n_b = B_pad // TB
    x_tile_bytes = TB * C * _round_up(HW, 128) * 2
    weight_bytes = (C * H_pad + H_pad + H_pad * NC_pad + NC_pad) * 4
    vmem_limit = min(2 * x_tile_bytes + 2 * weight_bytes
                     + TB * NC_pad * 4 + (8 << 20), 100 << 20)

    cost = pl.CostEstimate(
        flops=B_pad * C * HW + 2 * B_pad * (C * H_pad + H_pad * NC_pad),
        transcendentals=0,
        bytes_accessed=(B_pad * C * HW * 2 + weight_bytes + B_pad * NC_pad * 4),
    )

    out_padded = pl.pallas_call(
        functools.partial(_head_kernel, inv_hw=1.0 / float(HW)),
        out_shape=jax.ShapeDtypeStruct((B_pad, NC_pad), jnp.float32),
        grid=(n_b,),
        in_specs=[
            pl.BlockSpec((TB, C, HW), lambda i: (i, 0, 0)),
            pl.BlockSpec((C, H_pad), lambda i: (0, 0)),
            pl.BlockSpec((1, H_pad), lambda i: (0, 0)),
            pl.BlockSpec((H_pad, NC_pad), lambda i: (0, 0)),
            pl.BlockSpec((1, NC_pad), lambda i: (0, 0)),
        ],
        out_specs=pl.BlockSpec((TB, NC_pad), lambda i: (i, 0)),
        compiler_params=pltpu.CompilerParams(
            dimension_semantics=("parallel",),
            vmem_limit_bytes=vmem_limit,
        ),
        cost_estimate=cost,
    )(xr, w1t, b1_row, w2t, b2_row)

    return out_padded[:B, :NC]


# speedup vs baseline: 9.4290x; 4.0327x over previous
import functools

import jax
import jax.numpy as jnp
from jax.experimental import pallas as pl
from jax.experimental.pallas import tpu as pltpu


def _round_up(x, m):
    return ((x + m - 1) // m) * m


def _head_kernel(x_ref, w1t_ref, b1_ref, w2t_ref, b2_ref, out_ref, *, inv_hw):
    x = x_ref[...]
    pooled = jnp.sum(x, axis=1) * inv_hw
    h = jnp.dot(pooled, w1t_ref[...],
                preferred_element_type=jnp.float32) + b1_ref[...]
    out = jnp.dot(h, w2t_ref[...],
                  preferred_element_type=jnp.float32) + b2_ref[...]
    out_ref[...] = out.astype(out_ref.dtype)


def kernel(x, w1, b1, w2, b2):
    B, C, H, W = x.shape
    hidden = w1.shape[0]
    NC = w2.shape[0]
    HW = H * W

    TB = 8
    B_pad = _round_up(max(B, TB), TB)
    H_pad = _round_up(hidden, 128)
    NC_pad = _round_up(NC, 128)

    xr = jnp.transpose(x, (0, 2, 3, 1)).reshape(B, HW, C)
    if B_pad != B:
        xr = jnp.pad(xr, ((0, B_pad - B), (0, 0), (0, 0)))

    w1t = jnp.pad(w1.T, ((0, 0), (0, H_pad - hidden)))
    b1_row = jnp.pad(b1.reshape(1, -1), ((0, 0), (0, H_pad - hidden)))
    w2t = jnp.pad(w2.T, ((0, H_pad - hidden), (0, NC_pad - NC)))
    b2_row = jnp.pad(b2.reshape(1, -1), ((0, 0), (0, NC_pad - NC)))

    n_b = B_pad // TB
    x_tile_bytes = TB * _round_up(HW, 8) * C * 4
    weight_bytes = (C * H_pad + H_pad + H_pad * NC_pad + NC_pad) * 4
    vmem_limit = min(2 * x_tile_bytes + 2 * weight_bytes
                     + TB * NC_pad * 4 + (8 << 20), 100 << 20)

    cost = pl.CostEstimate(
        flops=B_pad * C * HW + 2 * B_pad * (C * H_pad + H_pad * NC_pad),
        transcendentals=0,
        bytes_accessed=(B_pad * C * HW * 4 + weight_bytes + B_pad * NC_pad * 4),
    )

    out_padded = pl.pallas_call(
        functools.partial(_head_kernel, inv_hw=1.0 / float(HW)),
        out_shape=jax.ShapeDtypeStruct((B_pad, NC_pad), jnp.float32),
        grid=(n_b,),
        in_specs=[
            pl.BlockSpec((TB, HW, C), lambda i: (i, 0, 0)),
            pl.BlockSpec((C, H_pad), lambda i: (0, 0)),
            pl.BlockSpec((1, H_pad), lambda i: (0, 0)),
            pl.BlockSpec((H_pad, NC_pad), lambda i: (0, 0)),
            pl.BlockSpec((1, NC_pad), lambda i: (0, 0)),
        ],
        out_specs=pl.BlockSpec((TB, NC_pad), lambda i: (i, 0)),
        compiler_params=pltpu.CompilerParams(
            dimension_semantics=("parallel",),
            vmem_limit_bytes=vmem_limit,
        ),
        cost_estimate=cost,
    )(xr, w1t, b1_row, w2t, b2_row)

    return out_padded[:B, :NC]
